# Initial kernel scaffold; baseline (speedup 1.0000x reference)
import jax
import jax.numpy as jnp
from jax.experimental import pallas as pl


def kernel(prediction, target, class_weights):
    raise NotImplementedError("write your pallas kernel here")


# speedup vs baseline: 193.6166x; 193.6166x over previous
import functools
import jax
import jax.numpy as jnp
from jax import lax
from jax.experimental import pallas as pl
from jax.experimental.pallas import tpu as pltpu

IGNORE = 255
H = W = 400
NPIX = H * W
K_FRAC = 0.25
K = int(K_FRAC * NPIX)
FMAX_BITS = 0x7F800000


def _body(pred_ref, tgt_ref, cw_ref, out_ref):
    c = pl.program_id(0)
    i = pl.program_id(1)
    p0 = pred_ref[0, 0, 0]
    p1 = pred_ref[0, 0, 1]
    t = tgt_ref[0, 0]
    valid = t != IGNORE
    is1 = t == 1
    d = jnp.where(is1, p0 - p1, p1 - p0)
    sp = jnp.maximum(d, 0.0) + jnp.log(1.0 + jnp.exp(-jnp.abs(d)))
    w = jnp.where(is1, cw_ref[c, 1], cw_ref[c, 0])
    loss = jnp.where(valid, w * sp, 0.0)

    @pl.when(c == 2)
    def _plain():
        out_ref[c, i] = jnp.sum(loss)

    @pl.when(c < 2)
    def _topk():
        bits = lax.bitcast_convert_type(loss, jnp.int32)

        def step(_, lohi):
            lo, hi = lohi
            mid = lo + (hi - lo + 1) // 2
            cnt = jnp.sum((bits > mid).astype(jnp.int32))
            take = cnt >= K
            return jnp.where(take, mid, lo), jnp.where(take, hi, mid)

        _, hi = lax.fori_loop(0, 31, step, (jnp.int32(-1), jnp.int32(FMAX_BITS)))
        thr = lax.bitcast_convert_type(hi, jnp.float32)
        gt = bits > hi
        cgt = jnp.sum(gt.astype(jnp.int32))
        sgt = jnp.sum(jnp.where(gt, loss, 0.0))
        out_ref[c, i] = sgt + (K - cgt).astype(jnp.float32) * thr


@jax.jit
def kernel(prediction, target, class_weights):
    b = prediction.shape[0]
    pred5 = prediction.reshape(b, 3, 2, H, W)
    sums = pl.pallas_call(
        _body,
        grid=(3, b),
        in_specs=[
            pl.BlockSpec((1, 1, 2, H, W), lambda c, i: (i, c, 0, 0, 0)),
            pl.BlockSpec((1, 1, H, W), lambda c, i: (i, c, 0, 0)),
            pl.BlockSpec(memory_space=pltpu.SMEM),
        ],
        out_specs=pl.BlockSpec((3, b), lambda c, i: (0, 0),
                               memory_space=pltpu.SMEM),
        out_shape=jax.ShapeDtypeStruct((3, b), jnp.float32),
    )(pred5, target, class_weights)
    total = (jnp.sum(sums[0]) / (b * K)
             + jnp.sum(sums[1]) / (b * K)
             + jnp.sum(sums[2]) / (b * NPIX))
    return total
